# Initial kernel scaffold; baseline (speedup 1.0000x reference)
import jax
import jax.numpy as jnp
from jax.experimental import pallas as pl


def kernel(x1, x2):
    raise NotImplementedError("write your pallas kernel here")


# speedup vs baseline: 545.9404x; 545.9404x over previous
import jax
import jax.numpy as jnp
from jax.experimental import pallas as pl

MARGIN = 1.0


def _loss_kernel(x1_ref, x2_ref, out_ref):
    a = x1_ref[:, :]
    b = x2_ref[:, :]
    n1 = a.shape[0]
    n2 = b.shape[0]

    na = jnp.sum(a * a, axis=1)
    nb = jnp.sum(b * b, axis=1)
    sa = jnp.sum(a, axis=0)
    sb = jnp.sum(b, axis=0)

    pos = (n1 * jnp.sum(na) - jnp.sum(sa * sa)
           + n2 * jnp.sum(nb) - jnp.sum(sb * sb))

    g = jax.lax.dot_general(a, b, (((1,), (1,)), ((), ())),
                            preferred_element_type=jnp.float32)
    d2 = na[:, None] + nb[None, :] - 2.0 * g
    d = jnp.sqrt(jnp.maximum(d2, 0.0))
    hinge = jnp.maximum(MARGIN - d, 0.0)
    neg = jnp.sum(hinge * hinge)

    n_pairs = n1 * (n1 - 1) // 2 + n2 * (n2 - 1) // 2 + n1 * n2
    out_ref[...] = ((pos + neg) / n_pairs).reshape(1, 1)


def kernel(x1, x2):
    out = pl.pallas_call(
        _loss_kernel,
        out_shape=jax.ShapeDtypeStruct((1, 1), jnp.float32),
    )(x1, x2)
    return out.reshape(())
